# baseline (speedup 1.0000x reference)
import jax
import jax.numpy as jnp
from jax import lax
from jax.experimental import pallas as pl
from jax.experimental.pallas import tpu as pltpu
from jax.experimental.pallas import tpu_sc as plsc

SEQ_LEN = 200
EMB_DIM = 128
LANES = 16
ROWS = 16
HALF = ROWS // 2
TAIL_TILE = 12
TAIL_BASE = SEQ_LEN - ROWS
ACTIVE_TILES = 13
ZERO_TILE = 15
CHUNKS = EMB_DIM // LANES

_mesh = plsc.VectorSubcoreMesh(
    core_axis_name="c", subcore_axis_name="s", num_cores=1
)


def _sc_body(idx_hbm, table_hbm, f_hbm, zi_hbm, out_hbm,
             idx_v, rows_v, f_v, part_v, zi_v, shared,
             sem_i, sem_f, sem_g, sem_z):
    sid = lax.axis_index("s")
    base = jnp.where(sid < TAIL_TILE, sid * ROWS,
                     jnp.where(sid == TAIL_TILE, TAIL_BASE, 0))

    @pl.when(sid == ZERO_TILE)
    def _():
        for c in range(CHUNKS):
            part_v[0, pl.ds(c * LANES, LANES)] = jnp.zeros((LANES,), jnp.float32)
        pltpu.sync_copy(part_v, shared)
        plsc.subcore_barrier()

    @pl.when(sid != ZERO_TILE)
    def _():
        cp_i = pltpu.async_copy(idx_hbm.at[pl.ds(base, ROWS)], idx_v, sem_i)
        cp_f = pltpu.async_copy(f_hbm.at[pl.ds(base, ROWS)], f_v, sem_f)
        cp_z = pltpu.async_copy(zi_hbm, zi_v, sem_z)
        plsc.subcore_barrier()

        cp_i.wait()
        cp_g = pltpu.async_copy(table_hbm.at[idx_v], rows_v, sem_g)
        cp_f.wait()
        cp_g.wait()

        def fma_row(r, accs):
            out = []
            for c in range(CHUNKS):
                col = pl.ds(c * LANES, LANES)
                out.append(accs[c] + rows_v[r, col] * f_v[r, col])
            return tuple(out)

        lo = jnp.where(sid == TAIL_TILE, HALF, 0)
        zero16 = jnp.zeros((LANES,), jnp.float32)
        accs = lax.fori_loop(lo, ROWS, fma_row, (zero16,) * CHUNKS)
        for c in range(CHUNKS):
            part_v[0, pl.ds(c * LANES, LANES)] = accs[c]
        cp_z.wait()

        @pl.when(sid < ACTIVE_TILES)
        def _():
            pltpu.sync_copy(part_v, shared.at[zi_v], add=True)

    plsc.subcore_barrier()

    @pl.when(sid == 0)
    def _():
        pltpu.sync_copy(shared, out_hbm)


_sc_call = pl.kernel(
    _sc_body,
    out_type=jax.ShapeDtypeStruct((1, EMB_DIM), jnp.float32),
    mesh=_mesh,
    scratch_types=[
        pltpu.VMEM((ROWS,), jnp.int32),
        pltpu.VMEM((ROWS, EMB_DIM), jnp.float32),
        pltpu.VMEM((ROWS, EMB_DIM), jnp.float32),
        pltpu.VMEM((1, EMB_DIM), jnp.float32),
        pltpu.VMEM((1,), jnp.int32),
        pltpu.VMEM_SHARED((1, EMB_DIM), jnp.float32),
        pltpu.SemaphoreType.DMA,
        pltpu.SemaphoreType.DMA,
        pltpu.SemaphoreType.DMA,
        pltpu.SemaphoreType.DMA,
    ],
)


def kernel(input_sequence, emb_table, f):
    zero_idx = jnp.zeros((1,), jnp.int32)
    return _sc_call(input_sequence.astype(jnp.int32), emb_table, f, zero_idx)


# speedup vs baseline: 1.0073x; 1.0073x over previous
import jax
import jax.numpy as jnp
from jax import lax
from jax.experimental import pallas as pl
from jax.experimental.pallas import tpu as pltpu
from jax.experimental.pallas import tpu_sc as plsc

SEQ_LEN = 200
EMB_DIM = 128
LANES = 16
ROWS = 16
HALF = ROWS // 2
TAIL_TILE = 12
TAIL_BASE = SEQ_LEN - ROWS
ACTIVE_TILES = 13
ZERO_TILE = 15
CHUNKS = EMB_DIM // LANES

_mesh = plsc.VectorSubcoreMesh(
    core_axis_name="c", subcore_axis_name="s", num_cores=1
)


def _sc_body(idx_hbm, table_hbm, f_hbm, zf_hbm, zi_hbm, out_hbm,
             idx_v, rows_v, f_v, part_v, zi_v, shared,
             sem_i, sem_f, sem_g, sem_z):
    sid = lax.axis_index("s")
    base = jnp.where(sid < TAIL_TILE, sid * ROWS,
                     jnp.where(sid == TAIL_TILE, TAIL_BASE, 0))

    @pl.when(sid == ZERO_TILE)
    def _():
        pltpu.sync_copy(zf_hbm, shared)
        plsc.subcore_barrier()

    @pl.when((sid >= ACTIVE_TILES) & (sid != ZERO_TILE))
    def _():
        plsc.subcore_barrier()

    @pl.when(sid < ACTIVE_TILES)
    def _():
        cp_i = pltpu.async_copy(idx_hbm.at[pl.ds(base, ROWS)], idx_v, sem_i)
        cp_f = pltpu.async_copy(f_hbm.at[pl.ds(base, ROWS)], f_v, sem_f)
        cp_z = pltpu.async_copy(zi_hbm, zi_v, sem_z)
        cp_i.wait()
        cp_g = pltpu.async_copy(table_hbm.at[idx_v], rows_v, sem_g)
        plsc.subcore_barrier()
        cp_f.wait()
        cp_g.wait()

        def fma_row(r, accs):
            out = []
            for c in range(CHUNKS):
                col = pl.ds(c * LANES, LANES)
                out.append(accs[c] + rows_v[r, col] * f_v[r, col])
            return tuple(out)

        lo = jnp.where(sid == TAIL_TILE, HALF, 0)
        zero16 = jnp.zeros((LANES,), jnp.float32)
        accs = lax.fori_loop(lo, ROWS, fma_row, (zero16,) * CHUNKS)
        for c in range(CHUNKS):
            part_v[0, pl.ds(c * LANES, LANES)] = accs[c]
        cp_z.wait()
        pltpu.sync_copy(part_v, shared.at[zi_v], add=True)

    plsc.subcore_barrier()

    @pl.when(sid == 0)
    def _():
        pltpu.sync_copy(shared, out_hbm)


_sc_call = pl.kernel(
    _sc_body,
    out_type=jax.ShapeDtypeStruct((1, EMB_DIM), jnp.float32),
    mesh=_mesh,
    scratch_types=[
        pltpu.VMEM((ROWS,), jnp.int32),
        pltpu.VMEM((ROWS, EMB_DIM), jnp.float32),
        pltpu.VMEM((ROWS, EMB_DIM), jnp.float32),
        pltpu.VMEM((1, EMB_DIM), jnp.float32),
        pltpu.VMEM((1,), jnp.int32),
        pltpu.VMEM_SHARED((1, EMB_DIM), jnp.float32),
        pltpu.SemaphoreType.DMA,
        pltpu.SemaphoreType.DMA,
        pltpu.SemaphoreType.DMA,
        pltpu.SemaphoreType.DMA,
    ],
)


def kernel(input_sequence, emb_table, f):
    zero_row = jnp.zeros((1, EMB_DIM), jnp.float32)
    zero_idx = jnp.zeros((1,), jnp.int32)
    return _sc_call(input_sequence.astype(jnp.int32), emb_table, f,
                    zero_row, zero_idx)
